# Initial kernel scaffold; baseline (speedup 1.0000x reference)
import jax
import jax.numpy as jnp
from jax.experimental import pallas as pl


def kernel(atom_features, membership, W1, b1, W2, b2):
    raise NotImplementedError("write your pallas kernel here")


# speedup vs baseline: 5.3123x; 5.3123x over previous
import functools

import jax
import jax.numpy as jnp
from jax import lax
from jax.experimental import pallas as pl
from jax.experimental.pallas import tpu as pltpu
from jax.experimental.pallas import tpu_sc as plsc

N_ATOMS = 320000
N_GRAPHS = 10000
FEAT = 128
HIDDEN = 100

_NC = 2
_NS = 16
_NW = _NC * _NS
_SB = 256
_NSB = N_ATOMS // _SB
_N_ITERS = (_NSB + _NW - 1) // _NW
_RPT = 624
_IDX_CH = 128
_CH = _SB // _IDX_CH


def _make_segsum():
    mesh = plsc.VectorSubcoreMesh(core_axis_name="c", subcore_axis_name="s")

    @functools.partial(
        pl.kernel,
        mesh=mesh,
        out_type=jax.ShapeDtypeStruct((_NC * N_GRAPHS, FEAT), jnp.float32),
        scratch_types=[
            pltpu.VMEM((_SB, FEAT), jnp.float32),
            pltpu.VMEM((_CH, _IDX_CH), jnp.int32),
            pltpu.VMEM_SHARED((N_GRAPHS, FEAT), jnp.float32),
        ],
    )
    def segsum(af_hbm, mem_hbm, out_hbm, rows_v, idx_v, acc_sh):
        c = lax.axis_index("c")
        s = lax.axis_index("s")
        wid = c * _NS + s

        zero16 = jnp.zeros((16,), jnp.float32)

        def zbody(i, carry):
            for j in range(FEAT // 16):
                rows_v[i, pl.ds(j * 16, 16)] = zero16
            return carry

        lax.fori_loop(0, _SB, zbody, 0)
        r0 = s * _RPT
        nfull = _RPT // _SB
        rem = _RPT - nfull * _SB
        for z in range(nfull):
            pltpu.sync_copy(rows_v, acc_sh.at[pl.ds(r0 + z * _SB, _SB)])
        pltpu.sync_copy(rows_v.at[pl.ds(0, rem)],
                        acc_sh.at[pl.ds(r0 + nfull * _SB, rem)])

        @pl.when(s == _NS - 1)
        def _():
            pltpu.sync_copy(rows_v.at[pl.ds(0, 16)],
                            acc_sh.at[pl.ds(r0 + _RPT, 16)])

        plsc.subcore_barrier()

        def body(i, carry):
            g = wid + i * _NW

            @pl.when(g < _NSB)
            def _():
                pltpu.sync_copy(af_hbm.at[pl.ds(g * _SB, _SB)], rows_v)
                pltpu.sync_copy(mem_hbm.at[g], idx_v)
                for j in range(_CH):
                    pltpu.sync_copy(rows_v.at[pl.ds(j * _IDX_CH, _IDX_CH)],
                                    acc_sh.at[idx_v.at[j]], add=True)

            return carry

        lax.fori_loop(0, _N_ITERS, body, 0)
        plsc.subcore_barrier()

        ob = c * N_GRAPHS + r0
        for z in range(nfull):
            pltpu.sync_copy(acc_sh.at[pl.ds(r0 + z * _SB, _SB)], rows_v)
            pltpu.sync_copy(rows_v, out_hbm.at[pl.ds(ob + z * _SB, _SB)])
        pltpu.sync_copy(acc_sh.at[pl.ds(r0 + nfull * _SB, rem)],
                        rows_v.at[pl.ds(0, rem)])
        pltpu.sync_copy(rows_v.at[pl.ds(0, rem)],
                        out_hbm.at[pl.ds(ob + nfull * _SB, rem)])

        @pl.when(s == _NS - 1)
        def _():
            pltpu.sync_copy(acc_sh.at[pl.ds(r0 + _RPT, 16)],
                            rows_v.at[pl.ds(0, 16)])
            pltpu.sync_copy(rows_v.at[pl.ds(0, 16)],
                            out_hbm.at[pl.ds(ob + _RPT, 16)])

    return segsum


_segsum = _make_segsum()

_MLP_BLK = 1000
_MLP_GRID = N_GRAPHS // _MLP_BLK


def _mlp_body(p0_ref, p1_ref, w1_ref, b1_ref, w2_ref, b2_ref, o_ref):
    g = p0_ref[...] + p1_ref[...]
    h = jnp.dot(g, w1_ref[...], preferred_element_type=jnp.float32)
    h = jnp.maximum(h + b1_ref[...], 0.0)
    o = jnp.dot(h, w2_ref[...], preferred_element_type=jnp.float32)
    o_ref[...] = jnp.maximum(o + b2_ref[...], 0.0)


def _mlp(partials, W1, b1, W2, b2):
    return pl.pallas_call(
        _mlp_body,
        grid=(_MLP_GRID,),
        in_specs=[
            pl.BlockSpec((_MLP_BLK, FEAT), lambda i: (i, 0)),
            pl.BlockSpec((_MLP_BLK, FEAT), lambda i: (i + _MLP_GRID, 0)),
            pl.BlockSpec((FEAT, HIDDEN), lambda i: (0, 0)),
            pl.BlockSpec((1, HIDDEN), lambda i: (0, 0)),
            pl.BlockSpec((HIDDEN, FEAT), lambda i: (0, 0)),
            pl.BlockSpec((1, FEAT), lambda i: (0, 0)),
        ],
        out_specs=pl.BlockSpec((_MLP_BLK, FEAT), lambda i: (i, 0)),
        out_shape=jax.ShapeDtypeStruct((N_GRAPHS, FEAT), jnp.float32),
    )(partials, partials, W1, b1.reshape(1, HIDDEN), W2, b2.reshape(1, FEAT))


def kernel(atom_features, membership, W1, b1, W2, b2):
    mem = membership.astype(jnp.int32).reshape(_NSB, _CH, _IDX_CH)
    partials = _segsum(atom_features, mem)
    return _mlp(partials, W1, b1, W2, b2)


# speedup vs baseline: 7.7248x; 1.4541x over previous
import functools

import jax
import jax.numpy as jnp
from jax import lax
from jax.experimental import pallas as pl
from jax.experimental.pallas import tpu as pltpu
from jax.experimental.pallas import tpu_sc as plsc

N_ATOMS = 320000
N_GRAPHS = 10000
FEAT = 128
HIDDEN = 100

_NC = 2
_NS = 16
_NW = _NC * _NS
_B = 128
_NB = N_ATOMS // _B
_NPAIR = (_NB // _NW + 2) // 2
_RPT = 624
_WCH = (128, 128, 128, 128, 112)


def _make_segsum():
    mesh = plsc.VectorSubcoreMesh(core_axis_name="c", subcore_axis_name="s")

    @functools.partial(
        pl.kernel,
        mesh=mesh,
        out_type=jax.ShapeDtypeStruct((_NC * N_GRAPHS, FEAT), jnp.float32),
        scratch_types=[
            pltpu.VMEM((2, _B, FEAT), jnp.float32),
            pltpu.VMEM((2, 1, _B), jnp.int32),
            pltpu.SemaphoreType.DMA,
            pltpu.SemaphoreType.DMA,
            pltpu.VMEM_SHARED((N_GRAPHS, FEAT), jnp.float32),
        ],
    )
    def segsum(af_hbm, mem_hbm, out_hbm, rows_v, idx_v, sem0, sem1, acc_sh):
        c = lax.axis_index("c")
        s = lax.axis_index("s")
        wid = c * _NS + s
        sems = (sem0, sem1)
        r0 = s * _RPT
        ob = c * N_GRAPHS + r0
        last = s == _NS - 1

        def rslice(b, n, m=_B):
            return rows_v.at[b] if n == m else rows_v.at[b, pl.ds(0, n)]

        def issue(k, b):
            g = wid + k * _NW

            @pl.when(g < _NB)
            def _():
                pltpu.async_copy(af_hbm.at[pl.ds(g * _B, _B)],
                                 rows_v.at[b], sems[b])
                pltpu.async_copy(mem_hbm.at[g], idx_v.at[b], sems[b])

        def consume(k, b):
            g = wid + k * _NW

            @pl.when(g < _NB)
            def _():
                pltpu.make_async_copy(af_hbm.at[pl.ds(g * _B, _B)],
                                      rows_v.at[b], sems[b]).wait()
                pltpu.make_async_copy(mem_hbm.at[g], idx_v.at[b],
                                      sems[b]).wait()
                pltpu.sync_copy(rows_v.at[b], acc_sh.at[idx_v.at[b, 0]],
                                add=True)

        issue(0, 0)

        zero16 = jnp.zeros((16,), jnp.float32)

        def zbody(i, carry):
            for j in range(FEAT // 16):
                rows_v[1, i, pl.ds(j * 16, 16)] = zero16
            return carry

        lax.fori_loop(0, _B, zbody, 0)

        off = 0
        for n in _WCH:
            pltpu.async_copy(rslice(1, n), acc_sh.at[pl.ds(r0 + off, n)],
                             sem1)
            off += n

        @pl.when(last)
        def _():
            pltpu.async_copy(rslice(1, 16), acc_sh.at[pl.ds(r0 + 624, 16)],
                             sem1)

        off = 0
        for n in _WCH:
            pltpu.make_async_copy(rslice(1, n),
                                  acc_sh.at[pl.ds(r0 + off, n)], sem1).wait()
            off += n

        @pl.when(last)
        def _():
            pltpu.make_async_copy(rslice(1, 16),
                                  acc_sh.at[pl.ds(r0 + 624, 16)],
                                  sem1).wait()

        plsc.subcore_barrier()

        def pair(kk, carry):
            k = kk * 2
            issue(k + 1, 1)
            consume(k, 0)
            issue(k + 2, 0)
            consume(k + 1, 1)
            return carry

        lax.fori_loop(0, _NPAIR, pair, 0)
        plsc.subcore_barrier()

        def st_dsc(z, n):
            b = z % 2
            return (rslice(b, n),
                    out_hbm.at[pl.ds(ob + z * _B, n)], sems[b])

        for z, n in enumerate(_WCH):
            if z >= 2:
                src, dst, sem = st_dsc(z - 2, _WCH[z - 2])
                pltpu.make_async_copy(src, dst, sem).wait()
            src, dst, sem = st_dsc(z, n)
            pltpu.sync_copy(acc_sh.at[pl.ds(r0 + z * _B, n)], rslice(z % 2, n))
            pltpu.async_copy(src, dst, sem)
        for z in (3, 4):
            src, dst, sem = st_dsc(z, _WCH[z])
            pltpu.make_async_copy(src, dst, sem).wait()

        @pl.when(last)
        def _():
            pltpu.sync_copy(acc_sh.at[pl.ds(r0 + 624, 16)], rslice(0, 16))
            pltpu.sync_copy(rslice(0, 16), out_hbm.at[pl.ds(ob + 624, 16)])

    return segsum


_segsum = _make_segsum()

_MLP_BLK = 1000
_MLP_GRID = N_GRAPHS // _MLP_BLK


def _mlp_body(p0_ref, p1_ref, w1_ref, b1_ref, w2_ref, b2_ref, o_ref):
    g = p0_ref[...] + p1_ref[...]
    h = jnp.dot(g, w1_ref[...], preferred_element_type=jnp.float32)
    h = jnp.maximum(h + b1_ref[...], 0.0)
    o = jnp.dot(h, w2_ref[...], preferred_element_type=jnp.float32)
    o_ref[...] = jnp.maximum(o + b2_ref[...], 0.0)


def _mlp(partials, W1, b1, W2, b2):
    return pl.pallas_call(
        _mlp_body,
        grid=(_MLP_GRID,),
        in_specs=[
            pl.BlockSpec((_MLP_BLK, FEAT), lambda i: (i, 0)),
            pl.BlockSpec((_MLP_BLK, FEAT), lambda i: (i + _MLP_GRID, 0)),
            pl.BlockSpec((FEAT, HIDDEN), lambda i: (0, 0)),
            pl.BlockSpec((1, HIDDEN), lambda i: (0, 0)),
            pl.BlockSpec((HIDDEN, FEAT), lambda i: (0, 0)),
            pl.BlockSpec((1, FEAT), lambda i: (0, 0)),
        ],
        out_specs=pl.BlockSpec((_MLP_BLK, FEAT), lambda i: (i, 0)),
        out_shape=jax.ShapeDtypeStruct((N_GRAPHS, FEAT), jnp.float32),
    )(partials, partials, W1, b1.reshape(1, HIDDEN), W2, b2.reshape(1, FEAT))


def kernel(atom_features, membership, W1, b1, W2, b2):
    mem = membership.astype(jnp.int32).reshape(_NB, 1, _B)
    partials = _segsum(atom_features, mem)
    return _mlp(partials, W1, b1, W2, b2)
